# Initial kernel scaffold; baseline (speedup 1.0000x reference)
import jax
import jax.numpy as jnp
from jax.experimental import pallas as pl


def kernel(x, params, edge_index, batch):
    raise NotImplementedError("write your pallas kernel here")


# speedup vs baseline: 2.8123x; 2.8123x over previous
import functools

import jax
import jax.numpy as jnp
from jax import lax
from jax.experimental import pallas as pl
from jax.experimental.pallas import tpu as pltpu
from jax.experimental.pallas import tpu_sc as plsc

N_NODES = 10000
N_EDGES = 160000
F_IN = 128
HID = 256
N_CLASSES = 10
N_GRAPHS = 128
N_LAYERS = 3

BM = 1000
F32 = jnp.float32

_SC_CORES = 2
_SC_SUBCORES = 16
_LANES = 16


def _dense_gin_first(h, ap0, ap1, p, relu_out):
    k = h.shape[1]
    ep = (1.0 + p['eps']).reshape(1, 1).astype(F32)

    def body(ep_ref, h_ref, a0_ref, a1_ref, wa_ref, ba_ref,
             wb_ref, bb_ref, o0_ref, o1_ref):
        e = ep_ref[0, 0]
        t = e * h_ref[...] + a0_ref[...] + a1_ref[...]
        mid = jnp.dot(t, wa_ref[...], preferred_element_type=F32) + ba_ref[...]
        mid = jnp.maximum(mid, 0.0)
        out = jnp.dot(mid, wb_ref[...], preferred_element_type=F32) + bb_ref[...]
        if relu_out:
            out = jnp.maximum(out, 0.0)
        o0_ref[...] = out[:, :HID // 2]
        o1_ref[...] = out[:, HID // 2:]

    hh = HID // 2
    return pl.pallas_call(
        body,
        grid=(N_NODES // BM,),
        in_specs=[
            pl.BlockSpec(memory_space=pltpu.SMEM),
            pl.BlockSpec((BM, k), lambda i: (i, 0)),
            pl.BlockSpec((BM, k), lambda i: (i, 0)),
            pl.BlockSpec((BM, k), lambda i: (i, 0)),
            pl.BlockSpec((k, HID), lambda i: (0, 0)),
            pl.BlockSpec((1, HID), lambda i: (0, 0)),
            pl.BlockSpec((HID, HID), lambda i: (0, 0)),
            pl.BlockSpec((1, HID), lambda i: (0, 0)),
        ],
        out_specs=[
            pl.BlockSpec((BM, hh), lambda i: (i, 0)),
            pl.BlockSpec((BM, hh), lambda i: (i, 0)),
        ],
        out_shape=[jax.ShapeDtypeStruct((N_NODES, hh), F32)] * 2,
    )(ep, h, ap0, ap1, p['Wa'], p['ba'].reshape(1, HID),
      p['Wb'], p['bb'].reshape(1, HID))


def _dense_gin(h0, h1, a0, a1, p, relu_out):
    kh = h0.shape[1]
    ep = (1.0 + p['eps']).reshape(1, 1).astype(F32)

    def body(ep_ref, h0_ref, h1_ref, a0_ref, a1_ref, wa_ref, ba_ref,
             wb_ref, bb_ref, o0_ref, o1_ref):
        e = ep_ref[0, 0]
        t0 = e * h0_ref[...] + a0_ref[...]
        t1 = e * h1_ref[...] + a1_ref[...]
        mid = (jnp.dot(t0, wa_ref[:kh, :], preferred_element_type=F32)
               + jnp.dot(t1, wa_ref[kh:, :], preferred_element_type=F32)
               + ba_ref[...])
        mid = jnp.maximum(mid, 0.0)
        out = jnp.dot(mid, wb_ref[...], preferred_element_type=F32) + bb_ref[...]
        if relu_out:
            out = jnp.maximum(out, 0.0)
        o0_ref[...] = out[:, :HID // 2]
        o1_ref[...] = out[:, HID // 2:]

    hh = HID // 2
    return pl.pallas_call(
        body,
        grid=(N_NODES // BM,),
        in_specs=[
            pl.BlockSpec(memory_space=pltpu.SMEM),
            pl.BlockSpec((BM, kh), lambda i: (i, 0)),
            pl.BlockSpec((BM, kh), lambda i: (i, 0)),
            pl.BlockSpec((BM, kh), lambda i: (i, 0)),
            pl.BlockSpec((BM, kh), lambda i: (i, 0)),
            pl.BlockSpec((2 * kh, HID), lambda i: (0, 0)),
            pl.BlockSpec((1, HID), lambda i: (0, 0)),
            pl.BlockSpec((HID, HID), lambda i: (0, 0)),
            pl.BlockSpec((1, HID), lambda i: (0, 0)),
        ],
        out_specs=[
            pl.BlockSpec((BM, hh), lambda i: (i, 0)),
            pl.BlockSpec((BM, hh), lambda i: (i, 0)),
        ],
        out_shape=[jax.ShapeDtypeStruct((N_NODES, hh), F32)] * 2,
    )(ep, h0, h1, a0, a1, p['Wa'], p['ba'].reshape(1, HID),
      p['Wb'], p['bb'].reshape(1, HID))


def _edge_ab(z0, z1, w1):
    hh = HID // 2
    w1a = w1[:HID, :]
    w1b = w1[HID:, :]

    def body(z0_ref, z1_ref, wa_ref, wb_ref, a_ref, b_ref):
        zz0 = z0_ref[...]
        zz1 = z1_ref[...]
        a_ref[...] = (jnp.dot(zz0, wa_ref[:hh, :], preferred_element_type=F32)
                      + jnp.dot(zz1, wa_ref[hh:, :], preferred_element_type=F32))
        b_ref[...] = (jnp.dot(zz0, wb_ref[:hh, :], preferred_element_type=F32)
                      + jnp.dot(zz1, wb_ref[hh:, :], preferred_element_type=F32))

    return pl.pallas_call(
        body,
        grid=(N_NODES // BM,),
        in_specs=[
            pl.BlockSpec((BM, hh), lambda i: (i, 0)),
            pl.BlockSpec((BM, hh), lambda i: (i, 0)),
            pl.BlockSpec((HID, HID), lambda i: (0, 0)),
            pl.BlockSpec((HID, HID), lambda i: (0, 0)),
        ],
        out_specs=[
            pl.BlockSpec((BM, HID), lambda i: (i, 0)),
            pl.BlockSpec((BM, HID), lambda i: (i, 0)),
        ],
        out_shape=[jax.ShapeDtypeStruct((N_NODES, HID), F32)] * 2,
    )(z0, z1, w1a, w1b)


def _node_mask_and_masked_x(z0, z1, x, mn):
    hh = HID // 2

    def body(z0_ref, z1_ref, x_ref, w1_ref, b1_ref, w2_ref, b2_ref,
             nm_ref, mx_ref):
        mid = (jnp.dot(z0_ref[...], w1_ref[:hh, :], preferred_element_type=F32)
               + jnp.dot(z1_ref[...], w1_ref[hh:, :], preferred_element_type=F32)
               + b1_ref[...])
        mid = jnp.maximum(mid, 0.0)
        v = jnp.dot(mid, w2_ref[...], preferred_element_type=F32) + b2_ref[...]
        nm = jax.nn.sigmoid(v)
        nm_ref[...] = nm
        mx_ref[...] = x_ref[...] * nm

    return pl.pallas_call(
        body,
        grid=(N_NODES // BM,),
        in_specs=[
            pl.BlockSpec((BM, hh), lambda i: (i, 0)),
            pl.BlockSpec((BM, hh), lambda i: (i, 0)),
            pl.BlockSpec((BM, F_IN), lambda i: (i, 0)),
            pl.BlockSpec((HID, HID), lambda i: (0, 0)),
            pl.BlockSpec((1, HID), lambda i: (0, 0)),
            pl.BlockSpec((HID, 1), lambda i: (0, 0)),
            pl.BlockSpec((1, 1), lambda i: (0, 0)),
        ],
        out_specs=[
            pl.BlockSpec((BM, 1), lambda i: (i, 0)),
            pl.BlockSpec((BM, F_IN), lambda i: (i, 0)),
        ],
        out_shape=[
            jax.ShapeDtypeStruct((N_NODES, 1), F32),
            jax.ShapeDtypeStruct((N_NODES, F_IN), F32),
        ],
    )(z0, z1, x, mn['W1'], mn['b1'].reshape(1, HID),
      mn['W2'], mn['b2'].reshape(1, 1))


def _sigmoid_tc(t):
    def body(t_ref, o_ref):
        o_ref[...] = jax.nn.sigmoid(t_ref[...])

    return pl.pallas_call(
        body,
        out_shape=jax.ShapeDtypeStruct(t.shape, F32),
    )(t)


def _pool_and_classify(z0, z1, m0, m1, batch_row, wc, bc):
    hh = HID // 2
    nblk = N_NODES // BM

    def body(b_ref, z0_ref, z1_ref, m0_ref, m1_ref, wc_ref, bc_ref,
             lg_ref, hs_ref, ho_ref, accz_ref, accm_ref, cnt_ref):
        i = pl.program_id(0)

        @pl.when(i == 0)
        def _init():
            accz_ref[...] = jnp.zeros_like(accz_ref)
            accm_ref[...] = jnp.zeros_like(accm_ref)
            cnt_ref[...] = jnp.zeros_like(cnt_ref)

        seg = b_ref[0]
        rows = lax.broadcasted_iota(jnp.int32, (N_GRAPHS, BM), 0)
        onehot_t = (rows == seg).astype(F32)
        ones = jnp.ones((BM, 1), F32)
        accz_ref[:, :hh] += jnp.dot(onehot_t, z0_ref[...],
                                    preferred_element_type=F32)
        accz_ref[:, hh:] += jnp.dot(onehot_t, z1_ref[...],
                                    preferred_element_type=F32)
        accm_ref[:, :hh] += jnp.dot(onehot_t, m0_ref[...],
                                    preferred_element_type=F32)
        accm_ref[:, hh:] += jnp.dot(onehot_t, m1_ref[...],
                                    preferred_element_type=F32)
        cnt_ref[...] += jnp.dot(onehot_t, ones, preferred_element_type=F32)

        @pl.when(i == nblk - 1)
        def _fin():
            den = jnp.maximum(cnt_ref[...], 1.0)
            ho = accz_ref[...] / den
            hs = accm_ref[...] / den
            ho_ref[...] = ho
            hs_ref[...] = hs
            lg_ref[...] = (jnp.dot(hs, wc_ref[...], preferred_element_type=F32)
                           + bc_ref[...])

    return pl.pallas_call(
        body,
        grid=(nblk,),
        in_specs=[
            pl.BlockSpec((1, 1, BM), lambda i: (i, 0, 0)),
            pl.BlockSpec((BM, hh), lambda i: (i, 0)),
            pl.BlockSpec((BM, hh), lambda i: (i, 0)),
            pl.BlockSpec((BM, hh), lambda i: (i, 0)),
            pl.BlockSpec((BM, hh), lambda i: (i, 0)),
            pl.BlockSpec((HID, N_CLASSES), lambda i: (0, 0)),
            pl.BlockSpec((1, N_CLASSES), lambda i: (0, 0)),
        ],
        out_specs=[
            pl.BlockSpec((N_GRAPHS, N_CLASSES), lambda i: (0, 0)),
            pl.BlockSpec((N_GRAPHS, HID), lambda i: (0, 0)),
            pl.BlockSpec((N_GRAPHS, HID), lambda i: (0, 0)),
        ],
        out_shape=[
            jax.ShapeDtypeStruct((N_GRAPHS, N_CLASSES), F32),
            jax.ShapeDtypeStruct((N_GRAPHS, HID), F32),
            jax.ShapeDtypeStruct((N_GRAPHS, HID), F32),
        ],
        scratch_shapes=[
            pltpu.VMEM((N_GRAPHS, HID), F32),
            pltpu.VMEM((N_GRAPHS, HID), F32),
            pltpu.VMEM((N_GRAPHS, 1), F32),
        ],
        compiler_params=pltpu.CompilerParams(
            dimension_semantics=("arbitrary",)),
    )(batch_row, z0, z1, m0, m1, wc, bc)


_SEG_CHUNK = 80
_N_SEG_CHUNKS = N_EDGES // _SEG_CHUNK
_ROW_SLAB = 640
_ROW_TAIL = N_NODES - 15 * _ROW_SLAB


def _row_slab_copy(sid, copy_fn):
    @pl.when(sid < _SC_SUBCORES - 1)
    def _main():
        copy_fn(sid * _ROW_SLAB, _ROW_SLAB)

    @pl.when(sid == _SC_SUBCORES - 1)
    def _tail():
        copy_fn(15 * _ROW_SLAB, _ROW_TAIL)


def _make_segsum(weighted, split_features):
    fw = HID // 2 if split_features else F_IN
    mesh = plsc.VectorSubcoreMesh(core_axis_name="c", subcore_axis_name="s",
                                  num_cores=_SC_CORES,
                                  num_subcores=_SC_SUBCORES)

    scratch = [
        pltpu.VMEM((_SEG_CHUNK,), jnp.int32),
        pltpu.VMEM((_SEG_CHUNK,), jnp.int32),
        pltpu.VMEM((_SEG_CHUNK, fw), F32),
        pltpu.VMEM((_SEG_CHUNK,), F32),
        pltpu.VMEM_SHARED((N_NODES, fw), F32),
    ]

    def body(h0_hbm, h1_hbm, src_hbm, dst_hbm, w_hbm, z_hbm,
             a0_hbm, a1_hbm, si_v, di_v, msg_v, w_v, acc_sh):
        cid = lax.axis_index("c")
        sid = lax.axis_index("s")

        _row_slab_copy(sid, lambda r0, nr: pltpu.sync_copy(
            z_hbm.at[pl.ds(r0, nr)], acc_sh.at[pl.ds(r0, nr)]))
        plsc.subcore_barrier()

        def chunk(ck):
            base = ck * _SEG_CHUNK
            pltpu.sync_copy(src_hbm.at[pl.ds(base, _SEG_CHUNK)], si_v)
            pltpu.sync_copy(dst_hbm.at[pl.ds(base, _SEG_CHUNK)], di_v)

            if split_features:
                @pl.when(cid == 0)
                def _g0():
                    pltpu.sync_copy(h0_hbm.at[si_v], msg_v)

                @pl.when(cid == 1)
                def _g1():
                    pltpu.sync_copy(h1_hbm.at[si_v], msg_v)
            else:
                pltpu.sync_copy(h0_hbm.at[si_v], msg_v)

            if weighted:
                pltpu.sync_copy(w_hbm.at[pl.ds(base, _SEG_CHUNK)], w_v)

                def scale(g, _):
                    wvec = w_v[pl.ds(g * _LANES, _LANES)]
                    for j in range(_LANES):
                        e = g * _LANES + j
                        for fg in range(fw // _LANES):
                            sl = pl.ds(fg * _LANES, _LANES)
                            msg_v[e, sl] = msg_v[e, sl] * wvec[j]
                    return _
                lax.fori_loop(0, _SEG_CHUNK // _LANES, scale, None)

            pltpu.sync_copy(msg_v, acc_sh.at[di_v], add=True)

        if split_features:
            def step(i, _):
                chunk(i * _SC_SUBCORES + sid)
                return _
            lax.fori_loop(0, _N_SEG_CHUNKS // _SC_SUBCORES, step, None)
        else:
            wid = sid * _SC_CORES + cid

            def step(i, _):
                ck = i * _N_WORKERS + wid

                @pl.when(ck < _N_SEG_CHUNKS)
                def _do():
                    chunk(ck)
                return _
            lax.fori_loop(0, -(-_N_SEG_CHUNKS // _N_WORKERS), step, None)

        plsc.subcore_barrier()

        @pl.when(cid == 0)
        def _w0():
            _row_slab_copy(sid, lambda r0, nr: pltpu.sync_copy(
                acc_sh.at[pl.ds(r0, nr)], a0_hbm.at[pl.ds(r0, nr)]))

        @pl.when(cid == 1)
        def _w1():
            _row_slab_copy(sid, lambda r0, nr: pltpu.sync_copy(
                acc_sh.at[pl.ds(r0, nr)], a1_hbm.at[pl.ds(r0, nr)]))

    return pl.kernel(
        body,
        out_type=[jax.ShapeDtypeStruct((N_NODES, fw), F32)] * 2,
        mesh=mesh,
        scratch_types=scratch,
        compiler_params=pltpu.CompilerParams(needs_layout_passes=False),
    )


def _segsum_call(weighted, h0, h1, src, dst, w):
    fn = _make_segsum(weighted, split_features=True)
    if w is None:
        w = jnp.zeros((N_EDGES,), F32)
    zeros = jnp.zeros((N_NODES, HID // 2), F32)
    return fn(h0, h1, src, dst, w, zeros)


def _segsum_first_call(weighted, h, src, dst, w):
    fn = _make_segsum(weighted, split_features=False)
    if w is None:
        w = jnp.zeros((N_EDGES,), F32)
    zeros = jnp.zeros((N_NODES, F_IN), F32)
    return fn(h, h, src, dst, w, zeros)


_EDGE_CHUNK = 128
_N_WORKERS = _SC_CORES * _SC_SUBCORES
_N_EDGE_CHUNKS = N_EDGES // _EDGE_CHUNK
_EDGE_STEPS = -(-_N_EDGE_CHUNKS // _N_WORKERS)


def _edge_logit_call(a, b, src, dst, b1, w2, b2):
    mesh = plsc.VectorSubcoreMesh(core_axis_name="c", subcore_axis_name="s",
                                  num_cores=_SC_CORES,
                                  num_subcores=_SC_SUBCORES)
    b2v = jnp.full((_LANES,), b2, F32)

    scratch = [
        pltpu.VMEM((_EDGE_CHUNK,), jnp.int32),
        pltpu.VMEM((_EDGE_CHUNK,), jnp.int32),
        pltpu.VMEM((_EDGE_CHUNK, HID), F32),
        pltpu.VMEM((_EDGE_CHUNK, HID), F32),
        pltpu.VMEM((_EDGE_CHUNK,), F32),
        pltpu.VMEM((HID,), F32),
        pltpu.VMEM((HID,), F32),
        pltpu.VMEM((_LANES,), F32),
        pltpu.VMEM((_LANES, _LANES), F32),
    ]

    def body(a_hbm, b_hbm, src_hbm, dst_hbm, b1_hbm, w2_hbm, b2_hbm,
             out_hbm, si_v, di_v, ar_v, br_v, o_v, b1_v, w2_v, b2_v, t16_v):
        cid = lax.axis_index("c")
        sid = lax.axis_index("s")
        wid = sid * _SC_CORES + cid
        lanes = lax.iota(jnp.int32, _LANES)

        pltpu.sync_copy(b1_hbm, b1_v)
        pltpu.sync_copy(w2_hbm, w2_v)
        pltpu.sync_copy(b2_hbm, b2_v)

        def step(i, _):
            ck = i * _N_WORKERS + wid

            @pl.when(ck < _N_EDGE_CHUNKS)
            def _do():
                base = ck * _EDGE_CHUNK
                pltpu.sync_copy(src_hbm.at[pl.ds(base, _EDGE_CHUNK)], si_v)
                pltpu.sync_copy(dst_hbm.at[pl.ds(base, _EDGE_CHUNK)], di_v)
                pltpu.sync_copy(a_hbm.at[si_v], ar_v)
                pltpu.sync_copy(b_hbm.at[di_v], br_v)
                b2vec = b2_v[...]

                def group(g, _):
                    for j in range(_LANES):
                        e = g * _LANES + j
                        acc = jnp.zeros((_LANES,), F32)
                        for fg in range(HID // _LANES):
                            sl = pl.ds(fg * _LANES, _LANES)
                            h = jnp.maximum(
                                ar_v[e, sl] + br_v[e, sl] + b1_v[sl], 0.0)
                            acc = acc + h * w2_v[sl]
                        t16_v[j, :] = acc
                    tot = b2vec
                    for k in range(_LANES):
                        col = plsc.load_gather(
                            t16_v, [lanes, jnp.full((_LANES,), k, jnp.int32)])
                        tot = tot + col
                    o_v[pl.ds(g * _LANES, _LANES)] = tot
                    return _
                lax.fori_loop(0, _EDGE_CHUNK // _LANES, group, None)
                pltpu.sync_copy(o_v, out_hbm.at[pl.ds(base, _EDGE_CHUNK)])
            return _

        lax.fori_loop(0, _EDGE_STEPS, step, None)

    fn = pl.kernel(
        body,
        out_type=jax.ShapeDtypeStruct((N_EDGES,), F32),
        mesh=mesh,
        scratch_types=scratch,
        compiler_params=pltpu.CompilerParams(needs_layout_passes=False),
    )
    return fn(a, b, src, dst, b1, w2.reshape(HID), b2v)


def _encoder_pass(x_full, src, dst, params, ew):
    weighted = ew is not None
    ap0, ap1 = _segsum_first_call(weighted, x_full, src, dst, ew)
    h0, h1 = _dense_gin_first(x_full, ap0, ap1, params['gin0'], relu_out=True)
    for i in range(1, N_LAYERS):
        a0, a1 = _segsum_call(weighted, h0, h1, src, dst, ew)
        h0, h1 = _dense_gin(h0, h1, a0, a1, params['gin%d' % i],
                            relu_out=(i < N_LAYERS - 1))
    return h0, h1


def kernel(x, params, edge_index, batch):
    x = x.astype(F32)
    src = edge_index[0]
    dst = edge_index[1]
    batch_row = batch.reshape(N_NODES // BM, 1, BM)

    z0, z1 = _encoder_pass(x, src, dst, params, None)

    nm, mx = _node_mask_and_masked_x(z0, z1, x, params['mn'])

    a, b = _edge_ab(z0, z1, params['me']['W1'])
    t = _edge_logit_call(a, b, src, dst, params['me']['b1'],
                         params['me']['W2'], params['me']['b2'][0])
    em2d = _sigmoid_tc(t.reshape(N_EDGES // 128, 128))
    em = em2d.reshape(N_EDGES)

    mz0, mz1 = _encoder_pass(mx, src, dst, params, em)

    logits, h_stable, h_orig = _pool_and_classify(
        z0, z1, mz0, mz1, batch_row, params['cls']['W'],
        params['cls']['b'].reshape(1, N_CLASSES))

    return (logits, h_stable, h_orig, nm, em.reshape(N_EDGES, 1))


# speedup vs baseline: 5.7181x; 2.0333x over previous
import functools

import jax
import jax.numpy as jnp
from jax import lax
from jax.experimental import pallas as pl
from jax.experimental.pallas import tpu as pltpu
from jax.experimental.pallas import tpu_sc as plsc

N_NODES = 10000
N_EDGES = 160000
F_IN = 128
HID = 256
N_CLASSES = 10
N_GRAPHS = 128
N_LAYERS = 3

BM = 1000
F32 = jnp.float32

_SC_CORES = 2
_SC_SUBCORES = 16
_LANES = 16


def _dense_gin_first(h, ap0, ap1, p, relu_out):
    k = h.shape[1]
    ep = (1.0 + p['eps']).reshape(1, 1).astype(F32)

    def body(ep_ref, h_ref, a0_ref, a1_ref, wa_ref, ba_ref,
             wb_ref, bb_ref, o0_ref, o1_ref):
        e = ep_ref[0, 0]
        t = e * h_ref[...] + a0_ref[...] + a1_ref[...]
        mid = jnp.dot(t, wa_ref[...], preferred_element_type=F32) + ba_ref[...]
        mid = jnp.maximum(mid, 0.0)
        out = jnp.dot(mid, wb_ref[...], preferred_element_type=F32) + bb_ref[...]
        if relu_out:
            out = jnp.maximum(out, 0.0)
        o0_ref[...] = out[:, :HID // 2]
        o1_ref[...] = out[:, HID // 2:]

    hh = HID // 2
    return pl.pallas_call(
        body,
        grid=(N_NODES // BM,),
        in_specs=[
            pl.BlockSpec(memory_space=pltpu.SMEM),
            pl.BlockSpec((BM, k), lambda i: (i, 0)),
            pl.BlockSpec((BM, k), lambda i: (i, 0)),
            pl.BlockSpec((BM, k), lambda i: (i, 0)),
            pl.BlockSpec((k, HID), lambda i: (0, 0)),
            pl.BlockSpec((1, HID), lambda i: (0, 0)),
            pl.BlockSpec((HID, HID), lambda i: (0, 0)),
            pl.BlockSpec((1, HID), lambda i: (0, 0)),
        ],
        out_specs=[
            pl.BlockSpec((BM, hh), lambda i: (i, 0)),
            pl.BlockSpec((BM, hh), lambda i: (i, 0)),
        ],
        out_shape=[jax.ShapeDtypeStruct((N_NODES, hh), F32)] * 2,
    )(ep, h, ap0, ap1, p['Wa'], p['ba'].reshape(1, HID),
      p['Wb'], p['bb'].reshape(1, HID))


def _dense_gin(h0, h1, a0, a1, p, relu_out):
    kh = h0.shape[1]
    ep = (1.0 + p['eps']).reshape(1, 1).astype(F32)

    def body(ep_ref, h0_ref, h1_ref, a0_ref, a1_ref, wa_ref, ba_ref,
             wb_ref, bb_ref, o0_ref, o1_ref):
        e = ep_ref[0, 0]
        t0 = e * h0_ref[...] + a0_ref[...]
        t1 = e * h1_ref[...] + a1_ref[...]
        mid = (jnp.dot(t0, wa_ref[:kh, :], preferred_element_type=F32)
               + jnp.dot(t1, wa_ref[kh:, :], preferred_element_type=F32)
               + ba_ref[...])
        mid = jnp.maximum(mid, 0.0)
        out = jnp.dot(mid, wb_ref[...], preferred_element_type=F32) + bb_ref[...]
        if relu_out:
            out = jnp.maximum(out, 0.0)
        o0_ref[...] = out[:, :HID // 2]
        o1_ref[...] = out[:, HID // 2:]

    hh = HID // 2
    return pl.pallas_call(
        body,
        grid=(N_NODES // BM,),
        in_specs=[
            pl.BlockSpec(memory_space=pltpu.SMEM),
            pl.BlockSpec((BM, kh), lambda i: (i, 0)),
            pl.BlockSpec((BM, kh), lambda i: (i, 0)),
            pl.BlockSpec((BM, kh), lambda i: (i, 0)),
            pl.BlockSpec((BM, kh), lambda i: (i, 0)),
            pl.BlockSpec((2 * kh, HID), lambda i: (0, 0)),
            pl.BlockSpec((1, HID), lambda i: (0, 0)),
            pl.BlockSpec((HID, HID), lambda i: (0, 0)),
            pl.BlockSpec((1, HID), lambda i: (0, 0)),
        ],
        out_specs=[
            pl.BlockSpec((BM, hh), lambda i: (i, 0)),
            pl.BlockSpec((BM, hh), lambda i: (i, 0)),
        ],
        out_shape=[jax.ShapeDtypeStruct((N_NODES, hh), F32)] * 2,
    )(ep, h0, h1, a0, a1, p['Wa'], p['ba'].reshape(1, HID),
      p['Wb'], p['bb'].reshape(1, HID))


def _edge_ab(z0, z1, w1):
    hh = HID // 2
    w1a = w1[:HID, :]
    w1b = w1[HID:, :]

    def body(z0_ref, z1_ref, wa_ref, wb_ref, a0_ref, a1_ref, b0_ref, b1_ref):
        zz0 = z0_ref[...]
        zz1 = z1_ref[...]
        av = (jnp.dot(zz0, wa_ref[:hh, :], preferred_element_type=F32)
              + jnp.dot(zz1, wa_ref[hh:, :], preferred_element_type=F32))
        bv = (jnp.dot(zz0, wb_ref[:hh, :], preferred_element_type=F32)
              + jnp.dot(zz1, wb_ref[hh:, :], preferred_element_type=F32))
        a0_ref[...] = av[:, :hh]
        a1_ref[...] = av[:, hh:]
        b0_ref[...] = bv[:, :hh]
        b1_ref[...] = bv[:, hh:]

    return pl.pallas_call(
        body,
        grid=(N_NODES // BM,),
        in_specs=[
            pl.BlockSpec((BM, hh), lambda i: (i, 0)),
            pl.BlockSpec((BM, hh), lambda i: (i, 0)),
            pl.BlockSpec((HID, HID), lambda i: (0, 0)),
            pl.BlockSpec((HID, HID), lambda i: (0, 0)),
        ],
        out_specs=[pl.BlockSpec((BM, hh), lambda i: (i, 0))] * 4,
        out_shape=[jax.ShapeDtypeStruct((N_NODES, hh), F32)] * 4,
    )(z0, z1, w1a, w1b)


def _node_mask_and_masked_x(z0, z1, x, mn):
    hh = HID // 2

    def body(z0_ref, z1_ref, x_ref, w1_ref, b1_ref, w2_ref, b2_ref,
             nm_ref, mx_ref):
        mid = (jnp.dot(z0_ref[...], w1_ref[:hh, :], preferred_element_type=F32)
               + jnp.dot(z1_ref[...], w1_ref[hh:, :], preferred_element_type=F32)
               + b1_ref[...])
        mid = jnp.maximum(mid, 0.0)
        v = jnp.dot(mid, w2_ref[...], preferred_element_type=F32) + b2_ref[...]
        nm = jax.nn.sigmoid(v)
        nm_ref[...] = nm
        mx_ref[...] = x_ref[...] * nm

    return pl.pallas_call(
        body,
        grid=(N_NODES // BM,),
        in_specs=[
            pl.BlockSpec((BM, hh), lambda i: (i, 0)),
            pl.BlockSpec((BM, hh), lambda i: (i, 0)),
            pl.BlockSpec((BM, F_IN), lambda i: (i, 0)),
            pl.BlockSpec((HID, HID), lambda i: (0, 0)),
            pl.BlockSpec((1, HID), lambda i: (0, 0)),
            pl.BlockSpec((HID, 1), lambda i: (0, 0)),
            pl.BlockSpec((1, 1), lambda i: (0, 0)),
        ],
        out_specs=[
            pl.BlockSpec((BM, 1), lambda i: (i, 0)),
            pl.BlockSpec((BM, F_IN), lambda i: (i, 0)),
        ],
        out_shape=[
            jax.ShapeDtypeStruct((N_NODES, 1), F32),
            jax.ShapeDtypeStruct((N_NODES, F_IN), F32),
        ],
    )(z0, z1, x, mn['W1'], mn['b1'].reshape(1, HID),
      mn['W2'], mn['b2'].reshape(1, 1))


def _sigmoid_tc(t0, t1, b2):
    b2a = b2.reshape(1, 1).astype(F32)

    def body(b2_ref, t0_ref, t1_ref, o_ref):
        o_ref[...] = jax.nn.sigmoid(t0_ref[...] + t1_ref[...] + b2_ref[0, 0])

    return pl.pallas_call(
        body,
        in_specs=[
            pl.BlockSpec(memory_space=pltpu.SMEM),
            pl.BlockSpec(t0.shape, lambda: (0, 0)),
            pl.BlockSpec(t0.shape, lambda: (0, 0)),
        ],
        out_specs=pl.BlockSpec(t0.shape, lambda: (0, 0)),
        out_shape=jax.ShapeDtypeStruct(t0.shape, F32),
    )(b2a, t0, t1)


def _pool_and_classify(z0, z1, m0, m1, batch_row, wc, bc):
    hh = HID // 2
    nblk = N_NODES // BM

    def body(b_ref, z0_ref, z1_ref, m0_ref, m1_ref, wc_ref, bc_ref,
             lg_ref, hs_ref, ho_ref, accz_ref, accm_ref, cnt_ref):
        i = pl.program_id(0)

        @pl.when(i == 0)
        def _init():
            accz_ref[...] = jnp.zeros_like(accz_ref)
            accm_ref[...] = jnp.zeros_like(accm_ref)
            cnt_ref[...] = jnp.zeros_like(cnt_ref)

        seg = b_ref[0]
        rows = lax.broadcasted_iota(jnp.int32, (N_GRAPHS, BM), 0)
        onehot_t = (rows == seg).astype(F32)
        ones = jnp.ones((BM, 1), F32)
        accz_ref[:, :hh] += jnp.dot(onehot_t, z0_ref[...],
                                    preferred_element_type=F32)
        accz_ref[:, hh:] += jnp.dot(onehot_t, z1_ref[...],
                                    preferred_element_type=F32)
        accm_ref[:, :hh] += jnp.dot(onehot_t, m0_ref[...],
                                    preferred_element_type=F32)
        accm_ref[:, hh:] += jnp.dot(onehot_t, m1_ref[...],
                                    preferred_element_type=F32)
        cnt_ref[...] += jnp.dot(onehot_t, ones, preferred_element_type=F32)

        @pl.when(i == nblk - 1)
        def _fin():
            den = jnp.maximum(cnt_ref[...], 1.0)
            ho = accz_ref[...] / den
            hs = accm_ref[...] / den
            ho_ref[...] = ho
            hs_ref[...] = hs
            lg_ref[...] = (jnp.dot(hs, wc_ref[...], preferred_element_type=F32)
                           + bc_ref[...])

    return pl.pallas_call(
        body,
        grid=(nblk,),
        in_specs=[
            pl.BlockSpec((1, 1, BM), lambda i: (i, 0, 0)),
            pl.BlockSpec((BM, hh), lambda i: (i, 0)),
            pl.BlockSpec((BM, hh), lambda i: (i, 0)),
            pl.BlockSpec((BM, hh), lambda i: (i, 0)),
            pl.BlockSpec((BM, hh), lambda i: (i, 0)),
            pl.BlockSpec((HID, N_CLASSES), lambda i: (0, 0)),
            pl.BlockSpec((1, N_CLASSES), lambda i: (0, 0)),
        ],
        out_specs=[
            pl.BlockSpec((N_GRAPHS, N_CLASSES), lambda i: (0, 0)),
            pl.BlockSpec((N_GRAPHS, HID), lambda i: (0, 0)),
            pl.BlockSpec((N_GRAPHS, HID), lambda i: (0, 0)),
        ],
        out_shape=[
            jax.ShapeDtypeStruct((N_GRAPHS, N_CLASSES), F32),
            jax.ShapeDtypeStruct((N_GRAPHS, HID), F32),
            jax.ShapeDtypeStruct((N_GRAPHS, HID), F32),
        ],
        scratch_shapes=[
            pltpu.VMEM((N_GRAPHS, HID), F32),
            pltpu.VMEM((N_GRAPHS, HID), F32),
            pltpu.VMEM((N_GRAPHS, 1), F32),
        ],
        compiler_params=pltpu.CompilerParams(
            dimension_semantics=("arbitrary",)),
    )(batch_row, z0, z1, m0, m1, wc, bc)


_N_WORKERS = _SC_CORES * _SC_SUBCORES
_ROW_SLAB = 640
_ROW_TAIL = N_NODES - 15 * _ROW_SLAB


def _row_slab_copy(sid, copy_fn):
    @pl.when(sid < _SC_SUBCORES - 1)
    def _main():
        copy_fn(sid * _ROW_SLAB, _ROW_SLAB)

    @pl.when(sid == _SC_SUBCORES - 1)
    def _tail():
        copy_fn(15 * _ROW_SLAB, _ROW_TAIL)


_C = 125
_NCH = N_EDGES // _C
_MROWS = 128
_S = 2
_IDXBLK = 8
_NGRP = _MROWS // _LANES


def _make_segsum(weighted, split_features):
    fw = HID // 2 if split_features else F_IN
    ncs = _NCH // _SC_SUBCORES if split_features else _NCH // _N_WORKERS
    mesh = plsc.VectorSubcoreMesh(core_axis_name="c", subcore_axis_name="s",
                                  num_cores=_SC_CORES,
                                  num_subcores=_SC_SUBCORES)

    scratch = [
        pltpu.VMEM((_IDXBLK, _C), jnp.int32),
        pltpu.VMEM((_IDXBLK, _C), jnp.int32),
        pltpu.VMEM_SHARED((N_NODES, fw), F32),
    ]
    scratch += [pltpu.VMEM((_MROWS, fw), F32) for _ in range(_S)]
    scratch += [pltpu.VMEM((_IDXBLK, _MROWS), F32)]
    scratch += [pltpu.SemaphoreType.DMA for _ in range(2 * _S)]

    def body(h0_hbm, h1_hbm, src_hbm, dst_hbm, w_hbm, z_hbm,
             a0_hbm, a1_hbm, si_v, di_v, acc_sh, *rest):
        msg = rest[:_S]
        w_blk = rest[_S]
        gsem = rest[_S + 1:_S + 1 + _S]
        ssem = rest[_S + 1 + _S:_S + 1 + 2 * _S]
        cid = lax.axis_index("c")
        sid = lax.axis_index("s")

        _row_slab_copy(sid, lambda r0, nr: pltpu.sync_copy(
            z_hbm.at[pl.ds(r0, nr)], acc_sh.at[pl.ds(r0, nr)]))

        base_ck = (sid * ncs if split_features
                   else (sid * _SC_CORES + cid) * ncs)
        plsc.subcore_barrier()

        def gather_desc(ck, b):
            dst = msg[b].at[pl.ds(0, _C)]
            if split_features:
                src0 = h0_hbm.at[si_v.at[ck]]
                src1 = h1_hbm.at[si_v.at[ck]]
                return (src0, src1, dst)
            return (h0_hbm.at[si_v.at[ck]], None, dst)

        def start_gather(ck, b):
            src0, src1, dst = gather_desc(ck, b)
            if src1 is None:
                pltpu.async_copy(src0, dst, gsem[b])
            else:
                @pl.when(cid == 0)
                def _g0():
                    pltpu.async_copy(src0, dst, gsem[b])

                @pl.when(cid == 1)
                def _g1():
                    pltpu.async_copy(src1, dst, gsem[b])

        def wait_gather(ck, b):
            src0, _, dst = gather_desc(ck, b)
            pltpu.make_async_copy(src0, dst, gsem[b]).wait()

        def blk(kb, _):
            blk0 = base_ck + kb * _IDXBLK
            pltpu.sync_copy(src_hbm.at[pl.ds(blk0, _IDXBLK)], si_v)
            pltpu.sync_copy(dst_hbm.at[pl.ds(blk0, _IDXBLK)], di_v)
            if weighted:
                pltpu.sync_copy(w_hbm.at[pl.ds(blk0, _IDXBLK)], w_blk)

            for b in range(_S):
                start_gather(b, b)

            def grp(g, _2):
                for b in range(_S):
                    ck = g * _S + b
                    wait_gather(ck, b)

                    if weighted:
                        def scale(sg, _3, b=b, ck=ck):
                            wvec = w_blk[ck, pl.ds(sg * _LANES, _LANES)]
                            for j in range(_LANES):
                                e = sg * _LANES + j
                                for fg in range(fw // _LANES):
                                    sl = pl.ds(fg * _LANES, _LANES)
                                    msg[b][e, sl] = msg[b][e, sl] * wvec[j]
                            return _3
                        lax.fori_loop(0, _NGRP, scale, None)

                    sc_src = msg[b].at[pl.ds(0, _C)]
                    sc_dst = acc_sh.at[di_v.at[ck]]
                    pltpu.async_copy(sc_src, sc_dst, ssem[b], add=True)
                    pltpu.make_async_copy(sc_src, sc_dst, ssem[b]).wait()

                    @pl.when(ck + _S < _IDXBLK)
                    def _next(ck=ck, b=b):
                        start_gather(ck + _S, b)
                return _2

            lax.fori_loop(0, _IDXBLK // _S, grp, None)
            return _

        lax.fori_loop(0, ncs // _IDXBLK, blk, None)
        plsc.subcore_barrier()

        @pl.when(cid == 0)
        def _w0():
            _row_slab_copy(sid, lambda r0, nr: pltpu.sync_copy(
                acc_sh.at[pl.ds(r0, nr)], a0_hbm.at[pl.ds(r0, nr)]))

        @pl.when(cid == 1)
        def _w1():
            _row_slab_copy(sid, lambda r0, nr: pltpu.sync_copy(
                acc_sh.at[pl.ds(r0, nr)], a1_hbm.at[pl.ds(r0, nr)]))

    return pl.kernel(
        body,
        out_type=[jax.ShapeDtypeStruct((N_NODES, fw), F32)] * 2,
        mesh=mesh,
        scratch_types=scratch,
        compiler_params=pltpu.CompilerParams(needs_layout_passes=False),
    )


def _segsum_call(weighted, h0, h1, src2d, dst2d, w2d):
    fn = _make_segsum(weighted, split_features=True)
    if w2d is None:
        w2d = jnp.zeros((_NCH, _MROWS), F32)
    zeros = jnp.zeros((N_NODES, HID // 2), F32)
    return fn(h0, h1, src2d, dst2d, w2d, zeros)


def _segsum_first_call(weighted, h, src2d, dst2d, w2d):
    fn = _make_segsum(weighted, split_features=False)
    if w2d is None:
        w2d = jnp.zeros((_NCH, _MROWS), F32)
    zeros = jnp.zeros((N_NODES, F_IN), F32)
    return fn(h, h, src2d, dst2d, w2d, zeros)


_ECS = _NCH // _SC_SUBCORES
_ES = 2
_HH = HID // 2


def _edge_logit_call(a0, a1, b0, b1h, src2d, dst2d, b1bias, w2):
    mesh = plsc.VectorSubcoreMesh(core_axis_name="c", subcore_axis_name="s",
                                  num_cores=_SC_CORES,
                                  num_subcores=_SC_SUBCORES)

    scratch = [
        pltpu.VMEM((_ECS, _C), jnp.int32),
        pltpu.VMEM((_ECS, _C), jnp.int32),
        pltpu.VMEM((_HH,), F32),
        pltpu.VMEM((_HH,), F32),
        pltpu.VMEM((_LANES, _LANES), F32),
        pltpu.VMEM((_ECS, _MROWS), F32),
    ]
    scratch += [pltpu.VMEM((_MROWS, _HH), F32) for _ in range(2 * _ES)]
    scratch += [pltpu.SemaphoreType.DMA for _ in range(_ES)]

    def body(a0_hbm, a1_hbm, b0_hbm, b1_hbm, src_hbm, dst_hbm,
             bias_hbm, w2_hbm, o0_hbm, o1_hbm,
             si_v, di_v, bb_v, w2_v, t16_v, ov_v, *rest):
        ar = rest[:_ES]
        br = rest[_ES:2 * _ES]
        gsem = rest[2 * _ES:3 * _ES]
        cid = lax.axis_index("c")
        sid = lax.axis_index("s")
        lanes = lax.iota(jnp.int32, _LANES)

        @pl.when(cid == 0)
        def _c0():
            pltpu.sync_copy(bias_hbm.at[pl.ds(0, _HH)], bb_v)
            pltpu.sync_copy(w2_hbm.at[pl.ds(0, _HH)], w2_v)

        @pl.when(cid == 1)
        def _c1():
            pltpu.sync_copy(bias_hbm.at[pl.ds(_HH, _HH)], bb_v)
            pltpu.sync_copy(w2_hbm.at[pl.ds(_HH, _HH)], w2_v)

        base_ck = sid * _ECS
        pltpu.sync_copy(src_hbm.at[pl.ds(base_ck, _ECS)], si_v)
        pltpu.sync_copy(dst_hbm.at[pl.ds(base_ck, _ECS)], di_v)

        def descs(ck, s):
            return (si_v.at[ck], ar[s].at[pl.ds(0, _C)],
                    di_v.at[ck], br[s].at[pl.ds(0, _C)])

        def start_gathers(ck, s):
            ia, da, ib, db = descs(ck, s)

            @pl.when(cid == 0)
            def _g0():
                pltpu.async_copy(a0_hbm.at[ia], da, gsem[s])
                pltpu.async_copy(b0_hbm.at[ib], db, gsem[s])

            @pl.when(cid == 1)
            def _g1():
                pltpu.async_copy(a1_hbm.at[ia], da, gsem[s])
                pltpu.async_copy(b1_hbm.at[ib], db, gsem[s])

        def wait_gathers(ck, s):
            ia, da, ib, db = descs(ck, s)
            pltpu.make_async_copy(a0_hbm.at[ia], da, gsem[s]).wait()
            pltpu.make_async_copy(b0_hbm.at[ib], db, gsem[s]).wait()

        for s in range(_ES):
            start_gathers(s, s)

        def grp(g, _):
            for s in range(_ES):
                ck = g * _ES + s
                wait_gathers(ck, s)

                def edge_grp(eg, _2, s=s):
                    for j in range(_LANES):
                        e = eg * _LANES + j
                        acc = jnp.zeros((_LANES,), F32)
                        for fg in range(_HH // _LANES):
                            sl = pl.ds(fg * _LANES, _LANES)
                            h = jnp.maximum(
                                ar[s][e, sl] + br[s][e, sl] + bb_v[sl], 0.0)
                            acc = acc + h * w2_v[sl]
                        t16_v[j, :] = acc
                    tot = jnp.zeros((_LANES,), F32)
                    for k in range(_LANES):
                        col = plsc.load_gather(
                            t16_v, [lanes, jnp.full((_LANES,), k, jnp.int32)])
                        tot = tot + col
                    ov_v[ck, pl.ds(eg * _LANES, _LANES)] = tot
                    return _2
                lax.fori_loop(0, _MROWS // _LANES, edge_grp, None)

                @pl.when(ck + _ES < _ECS)
                def _next(ck=ck, s=s):
                    start_gathers(ck + _ES, s)
            return _

        lax.fori_loop(0, _ECS // _ES, grp, None)

        @pl.when(cid == 0)
        def _w0():
            pltpu.sync_copy(ov_v, o0_hbm.at[pl.ds(base_ck, _ECS)])

        @pl.when(cid == 1)
        def _w1():
            pltpu.sync_copy(ov_v, o1_hbm.at[pl.ds(base_ck, _ECS)])

    fn = pl.kernel(
        body,
        out_type=[jax.ShapeDtypeStruct((_NCH, _MROWS), F32)] * 2,
        mesh=mesh,
        scratch_types=scratch,
        compiler_params=pltpu.CompilerParams(needs_layout_passes=False),
    )
    return fn(a0, a1, b0, b1h, src2d, dst2d, b1bias, w2.reshape(HID))


def _encoder_pass(x_full, src2d, dst2d, params, ew2d):
    weighted = ew2d is not None
    ap0, ap1 = _segsum_first_call(weighted, x_full, src2d, dst2d, ew2d)
    h0, h1 = _dense_gin_first(x_full, ap0, ap1, params['gin0'], relu_out=True)
    for i in range(1, N_LAYERS):
        a0, a1 = _segsum_call(weighted, h0, h1, src2d, dst2d, ew2d)
        h0, h1 = _dense_gin(h0, h1, a0, a1, params['gin%d' % i],
                            relu_out=(i < N_LAYERS - 1))
    return h0, h1


def kernel(x, params, edge_index, batch):
    x = x.astype(F32)
    src = edge_index[0]
    dst = edge_index[1]
    src2d = src.reshape(_NCH, _C)
    dst2d = dst.reshape(_NCH, _C)
    batch_row = batch.reshape(N_NODES // BM, 1, BM)

    z0, z1 = _encoder_pass(x, src2d, dst2d, params, None)

    nm, mx = _node_mask_and_masked_x(z0, z1, x, params['mn'])

    ea0, ea1, eb0, eb1 = _edge_ab(z0, z1, params['me']['W1'])
    t0, t1 = _edge_logit_call(ea0, ea1, eb0, eb1, src2d, dst2d,
                              params['me']['b1'], params['me']['W2'])
    em2d = _sigmoid_tc(t0, t1, params['me']['b2'][0])
    em = em2d[:, :_C].reshape(N_EDGES)

    mz0, mz1 = _encoder_pass(mx, src2d, dst2d, params, em2d)

    logits, h_stable, h_orig = _pool_and_classify(
        z0, z1, mz0, mz1, batch_row, params['cls']['W'],
        params['cls']['b'].reshape(1, N_CLASSES))

    return (logits, h_stable, h_orig, nm, em.reshape(N_EDGES, 1))
